# Initial kernel scaffold; baseline (speedup 1.0000x reference)
import jax
import jax.numpy as jnp
from jax.experimental import pallas as pl


def kernel(dense, id_list, offsets, tables, Wd, bd, Wp, bp, Wo, bo):
    raise NotImplementedError("write your pallas kernel here")


# speedup vs baseline: 1.7788x; 1.7788x over previous
import functools

import numpy as np
import jax
import jax.numpy as jnp
from jax import lax
from jax.experimental import pallas as pl
from jax.experimental.pallas import tpu as pltpu
from jax.experimental.pallas import tpu_sc as plsc

B = 4096
ND = 13
NF = 26
NFE = NF + 1
MH = 10000
H = 160
ED = 64

NC, NS = 2, 16
NW = NC * NS
NP = NF // 2
PW = 2 * ED
NROWS = NP * B
WPR = NROWS // NW
CH = 128
NCH = WPR // CH


@functools.partial(
    pl.kernel,
    mesh=plsc.VectorSubcoreMesh(core_axis_name="c", subcore_axis_name="s"),
    out_type=jax.ShapeDtypeStruct((NROWS, PW), jnp.float32),
    scratch_types=[
        pltpu.VMEM((NCH, CH), jnp.int32),
        pltpu.VMEM((CH, PW), jnp.float32),
        pltpu.VMEM((CH, PW), jnp.float32),
        pltpu.SemaphoreType.DMA,
        pltpu.SemaphoreType.DMA,
    ],
)
def _sc_gather(tab_ref, idx_ref, out_ref, idx_v, buf0, buf1, sem0, sem1):
    wid = lax.axis_index("s") * NC + lax.axis_index("c")
    base = wid * WPR
    pltpu.sync_copy(idx_ref.at[wid], idx_v)
    bufs = (buf0, buf1)
    sems = (sem0, sem1)
    cps = []
    for c in range(NCH):
        cps.append(pltpu.async_copy(tab_ref.at[idx_v.at[c]],
                                    bufs[c % 2], sems[c % 2]))
        if c >= 1:
            cps[c - 1].wait()
            pltpu.sync_copy(bufs[(c - 1) % 2],
                            out_ref.at[pl.ds(base + (c - 1) * CH, CH)])
    cps[NCH - 1].wait()
    pltpu.sync_copy(bufs[(NCH - 1) % 2],
                    out_ref.at[pl.ds(base + (NCH - 1) * CH, CH)])


TBLK = 2000


def _tab_body(t_ref, w_ref, b_ref, o_ref):
    halves = []
    for k in range(2):
        r = t_ref[0, k]
        n2 = jnp.sum(r * r, axis=1, keepdims=True)
        s = jnp.where(n2 > 1.0, lax.rsqrt(n2), 1.0)
        halves.append(
            jnp.dot(r * s, w_ref[0, k], preferred_element_type=jnp.float32)
            + b_ref[0, k]
        )
    o_ref[0] = jnp.concatenate(halves, axis=1)


def _table_transform(tables4, WpT4, bp4):
    return pl.pallas_call(
        _tab_body,
        grid=(NP, MH // TBLK),
        in_specs=[
            pl.BlockSpec((1, 2, TBLK, H), lambda p, i: (p, 0, i, 0)),
            pl.BlockSpec((1, 2, H, ED), lambda p, i: (p, 0, 0, 0)),
            pl.BlockSpec((1, 2, 1, ED), lambda p, i: (p, 0, 0, 0)),
        ],
        out_specs=pl.BlockSpec((1, TBLK, PW), lambda p, i: (p, i, 0)),
        out_shape=jax.ShapeDtypeStruct((NP, MH, PW), jnp.float32),
    )(tables4, WpT4, bp4)


BLK = 256
NPAIR = NFE * (NFE - 1) // 2
PADPAIR = 384


def _bat_body(g_ref, dt_ref, wd_ref, bd_ref, w1_ref, w2_ref, bo_ref, o_ref,
              ecat_ref, gt_ref):
    e0t = (
        jnp.dot(wd_ref[...], dt_ref[...], preferred_element_type=jnp.float32)
        + bd_ref[...]
    )
    ets = [e0t]
    for p in range(NP):
        tp = jnp.transpose(g_ref[p])
        ets.append(tp[:ED])
        ets.append(tp[ED:])
    for f in range(NFE):
        ecat_ref[f * ED:(f + 1) * ED, :] = ets[f]
    k = 0
    for i in range(NFE):
        for j in range(i + 1, NFE):
            gt_ref[k, :] = jnp.sum(ets[i] * ets[j], axis=0)
            k += 1
    gt_ref[NPAIR:PADPAIR, :] = jnp.zeros((PADPAIR - NPAIR, BLK), jnp.float32)
    outt = (
        jnp.dot(w1_ref[...], ecat_ref[...], preferred_element_type=jnp.float32)
        + jnp.dot(w2_ref[...], gt_ref[...], preferred_element_type=jnp.float32)
        + bo_ref[...]
    )
    o_ref[...] = jnp.transpose(outt)


def _batch_compute(g3, denseT, Wd, bd2, W1m, W2m, bo2):
    return pl.pallas_call(
        _bat_body,
        grid=(B // BLK,),
        in_specs=[
            pl.BlockSpec((NP, BLK, PW), lambda i: (0, i, 0)),
            pl.BlockSpec((ND, BLK), lambda i: (0, i)),
            pl.BlockSpec((ED, ND), lambda i: (0, 0)),
            pl.BlockSpec((ED, 1), lambda i: (0, 0)),
            pl.BlockSpec((ED, NFE * ED), lambda i: (0, 0)),
            pl.BlockSpec((ED, PADPAIR), lambda i: (0, 0)),
            pl.BlockSpec((ED, 1), lambda i: (0, 0)),
        ],
        out_specs=pl.BlockSpec((BLK, ED), lambda i: (i, 0)),
        out_shape=jax.ShapeDtypeStruct((B, ED), jnp.float32),
        scratch_shapes=[
            pltpu.VMEM((NFE * ED, BLK), jnp.float32),
            pltpu.VMEM((PADPAIR, BLK), jnp.float32),
        ],
    )(g3, denseT, Wd, bd2, W1m, W2m, bo2)


def kernel(dense, id_list, offsets, tables, Wd, bd, Wp, bp, Wo, bo):
    ids = (id_list.astype(jnp.int32)) % MH
    idx3 = (
        jnp.arange(NP, dtype=jnp.int32)[:, None] * MH + ids[None, :]
    ).reshape(NW, NCH, CH)
    WpT4 = jnp.transpose(Wp, (0, 2, 1)).reshape(NP, 2, H, ED)
    tproj = _table_transform(
        tables.reshape(NP, 2, MH, H), WpT4, bp.reshape(NP, 2, 1, ED))
    g = _sc_gather(tproj.reshape(NP * MH, PW), idx3)
    g3 = g.reshape(NP, B, PW)
    W1m = Wo[:, : NFE * ED]
    W2m = jnp.pad(Wo[:, NFE * ED:], ((0, 0), (0, PADPAIR - NPAIR)))
    return _batch_compute(g3, dense.T, Wd, bd[:, None], W1m, W2m, bo[:, None])


# speedup vs baseline: 9.7981x; 5.5082x over previous
import functools

import numpy as np
import jax
import jax.numpy as jnp
from jax import lax
from jax.experimental import pallas as pl
from jax.experimental.pallas import tpu as pltpu
from jax.experimental.pallas import tpu_sc as plsc

B = 4096
ND = 13
NF = 26
NFE = NF + 1
MH = 10000
H = 160
ED = 64

NC, NS = 2, 16
NW = NC * NS
NP = NF // 2
PW = 2 * ED
NROWS = NP * B
WPR = NROWS // NW
CH = 128
NCH = WPR // CH


@functools.partial(
    pl.kernel,
    mesh=plsc.VectorSubcoreMesh(core_axis_name="c", subcore_axis_name="s"),
    out_type=jax.ShapeDtypeStruct((NROWS, PW), jnp.float32),
    scratch_types=[
        pltpu.VMEM((NCH, CH), jnp.int32),
        pltpu.VMEM((CH, PW), jnp.float32),
        pltpu.VMEM((CH, PW), jnp.float32),
        pltpu.SemaphoreType.DMA,
        pltpu.SemaphoreType.DMA,
    ],
)
def _sc_gather(tab_ref, idx_ref, out_ref, idx_v, buf0, buf1, sem0, sem1):
    wid = lax.axis_index("s") * NC + lax.axis_index("c")
    base = wid * WPR
    pltpu.sync_copy(idx_ref.at[wid], idx_v)
    bufs = (buf0, buf1)
    sems = (sem0, sem1)
    cps = []
    for c in range(NCH):
        cps.append(pltpu.async_copy(tab_ref.at[idx_v.at[c]],
                                    bufs[c % 2], sems[c % 2]))
        if c >= 1:
            cps[c - 1].wait()
            pltpu.sync_copy(bufs[(c - 1) % 2],
                            out_ref.at[pl.ds(base + (c - 1) * CH, CH)])
    cps[NCH - 1].wait()
    pltpu.sync_copy(bufs[(NCH - 1) % 2],
                    out_ref.at[pl.ds(base + (NCH - 1) * CH, CH)])


TBLK = 1024


def _tab_body(t_ref, w_ref, b_ref, o_ref):
    halves = []
    for k in range(2):
        r = t_ref[0, k]
        n2 = jnp.sum(r * r, axis=0, keepdims=True)
        s = jnp.where(n2 > 1.0, lax.rsqrt(n2), 1.0)
        halves.append(
            jnp.dot(w_ref[0, k], r * s, preferred_element_type=jnp.float32)
            + b_ref[0, k]
        )
    o_ref[0] = jnp.transpose(jnp.concatenate(halves, axis=0))


def _table_transform(tablesT4, Wp4, bp4):
    return pl.pallas_call(
        _tab_body,
        grid=(NP, pl.cdiv(MH, TBLK)),
        in_specs=[
            pl.BlockSpec((1, 2, H, TBLK), lambda p, i: (p, 0, 0, i)),
            pl.BlockSpec((1, 2, ED, H), lambda p, i: (p, 0, 0, 0)),
            pl.BlockSpec((1, 2, ED, 1), lambda p, i: (p, 0, 0, 0)),
        ],
        out_specs=pl.BlockSpec((1, TBLK, PW), lambda p, i: (p, i, 0)),
        out_shape=jax.ShapeDtypeStruct((NP, MH, PW), jnp.float32),
    )(tablesT4, Wp4, bp4)


BLK = 256
NPAIR = NFE * (NFE - 1) // 2
PADPAIR = 384


def _bat_body(g_ref, dt_ref, wd_ref, bd_ref, w1_ref, w2_ref, bo_ref, o_ref,
              ecat_ref, gt_ref):
    e0t = (
        jnp.dot(wd_ref[...], dt_ref[...], preferred_element_type=jnp.float32)
        + bd_ref[...]
    )
    ets = [e0t]
    for p in range(NP):
        tp = jnp.transpose(g_ref[p])
        ets.append(tp[:ED])
        ets.append(tp[ED:])
    for f in range(NFE):
        ecat_ref[f * ED:(f + 1) * ED, :] = ets[f]
    k = 0
    for i in range(NFE):
        for j in range(i + 1, NFE):
            gt_ref[k, :] = jnp.sum(ets[i] * ets[j], axis=0)
            k += 1
    gt_ref[NPAIR:PADPAIR, :] = jnp.zeros((PADPAIR - NPAIR, BLK), jnp.float32)
    o_ref[...] = (
        jnp.dot(w1_ref[...], ecat_ref[...], preferred_element_type=jnp.float32)
        + jnp.dot(w2_ref[...], gt_ref[...], preferred_element_type=jnp.float32)
        + bo_ref[...]
    )


def _batch_compute(g3, denseT, Wd, bd2, W1m, W2m, bo2):
    return pl.pallas_call(
        _bat_body,
        grid=(B // BLK,),
        in_specs=[
            pl.BlockSpec((NP, BLK, PW), lambda i: (0, i, 0)),
            pl.BlockSpec((ND, BLK), lambda i: (0, i)),
            pl.BlockSpec((ED, ND), lambda i: (0, 0)),
            pl.BlockSpec((ED, 1), lambda i: (0, 0)),
            pl.BlockSpec((ED, NFE * ED), lambda i: (0, 0)),
            pl.BlockSpec((ED, PADPAIR), lambda i: (0, 0)),
            pl.BlockSpec((ED, 1), lambda i: (0, 0)),
        ],
        out_specs=pl.BlockSpec((ED, BLK), lambda i: (0, i)),
        out_shape=jax.ShapeDtypeStruct((ED, B), jnp.float32),
        scratch_shapes=[
            pltpu.VMEM((NFE * ED, BLK), jnp.float32),
            pltpu.VMEM((PADPAIR, BLK), jnp.float32),
        ],
    )(g3, denseT, Wd, bd2, W1m, W2m, bo2)


def kernel(dense, id_list, offsets, tables, Wd, bd, Wp, bp, Wo, bo):
    ids = (id_list.astype(jnp.int32)) % MH
    idx3 = (
        jnp.arange(NP, dtype=jnp.int32)[:, None] * MH + ids[None, :]
    ).reshape(NW, NCH, CH)
    tablesT4 = jnp.transpose(tables, (0, 2, 1)).reshape(NP, 2, H, MH)
    tproj = _table_transform(tablesT4, Wp.reshape(NP, 2, ED, H),
                             bp.reshape(NP, 2, ED, 1))
    g = _sc_gather(tproj.reshape(NP * MH, PW), idx3)
    g3 = g.reshape(NP, B, PW)
    W1m = Wo[:, : NFE * ED]
    W2m = jnp.pad(Wo[:, NFE * ED:], ((0, 0), (0, PADPAIR - NPAIR)))
    outT = _batch_compute(g3, dense.T, Wd, bd[:, None], W1m, W2m, bo[:, None])
    return outT.T


# speedup vs baseline: 11.9966x; 1.2244x over previous
import functools

import numpy as np
import jax
import jax.numpy as jnp
from jax import lax
from jax.experimental import pallas as pl
from jax.experimental.pallas import tpu as pltpu
from jax.experimental.pallas import tpu_sc as plsc

B = 4096
ND = 13
NF = 26
NFE = NF + 1
MH = 10000
H = 160
ED = 64

NC, NS = 2, 16
NW = NC * NS
NP = NF // 2
PW = 2 * ED
NROWS = NP * B
WPR = NROWS // NW
CH = 128
NCH = WPR // CH


@functools.partial(
    pl.kernel,
    mesh=plsc.VectorSubcoreMesh(core_axis_name="c", subcore_axis_name="s"),
    out_type=jax.ShapeDtypeStruct((NROWS, PW), jnp.float32),
    scratch_types=[
        pltpu.VMEM((NCH, CH), jnp.int32),
        pltpu.VMEM((CH, PW), jnp.float32),
        pltpu.VMEM((CH, PW), jnp.float32),
        pltpu.SemaphoreType.DMA,
        pltpu.SemaphoreType.DMA,
    ],
)
def _sc_gather(tab_ref, idx_ref, out_ref, idx_v, buf0, buf1, sem0, sem1):
    wid = lax.axis_index("s") * NC + lax.axis_index("c")
    base = wid * WPR
    pltpu.sync_copy(idx_ref.at[wid], idx_v)
    bufs = (buf0, buf1)
    sems = (sem0, sem1)
    cps = []
    for c in range(NCH):
        cps.append(pltpu.async_copy(tab_ref.at[idx_v.at[c]],
                                    bufs[c % 2], sems[c % 2]))
        if c >= 1:
            cps[c - 1].wait()
            pltpu.sync_copy(bufs[(c - 1) % 2],
                            out_ref.at[pl.ds(base + (c - 1) * CH, CH)])
    cps[NCH - 1].wait()
    pltpu.sync_copy(bufs[(NCH - 1) % 2],
                    out_ref.at[pl.ds(base + (NCH - 1) * CH, CH)])


TBLK = 2048


def _tab_body(t_ref, w_ref, b_ref, o_ref):
    halves = []
    for k in range(2):
        r = t_ref[0, k]
        n2 = jnp.sum(r * r, axis=0, keepdims=True)
        s = jnp.where(n2 > 1.0, lax.rsqrt(n2), 1.0)
        halves.append(
            jnp.dot(w_ref[0, k], r * s, preferred_element_type=jnp.float32)
            + b_ref[0, k]
        )
    o_ref[0] = jnp.transpose(jnp.concatenate(halves, axis=0))


def _table_transform(tablesT4, Wp4, bp4):
    return pl.pallas_call(
        _tab_body,
        grid=(NP, pl.cdiv(MH, TBLK)),
        in_specs=[
            pl.BlockSpec((1, 2, H, TBLK), lambda p, i: (p, 0, 0, i)),
            pl.BlockSpec((1, 2, ED, H), lambda p, i: (p, 0, 0, 0)),
            pl.BlockSpec((1, 2, ED, 1), lambda p, i: (p, 0, 0, 0)),
        ],
        out_specs=pl.BlockSpec((1, TBLK, PW), lambda p, i: (p, i, 0)),
        out_shape=jax.ShapeDtypeStruct((NP, MH, PW), jnp.float32),
    )(tablesT4, Wp4, bp4)


BLK = 256
NPAIR = NFE * (NFE - 1) // 2
PADPAIR = 384


def _bat_body(g_ref, dt_ref, wd_ref, bd_ref, w1_ref, w2_ref, bo_ref, o_ref,
              ecat_ref, gt_ref):
    e0t = (
        jnp.dot(wd_ref[...], dt_ref[...], preferred_element_type=jnp.float32)
        + bd_ref[...]
    )
    ets = [e0t]
    for p in range(NP):
        tp = jnp.transpose(g_ref[p])
        ets.append(tp[:ED])
        ets.append(tp[ED:])
    for f in range(NFE):
        ecat_ref[f * ED:(f + 1) * ED, :] = ets[f]
    k = 0
    for i in range(NFE):
        for j in range(i + 1, NFE):
            gt_ref[k, :] = jnp.sum(ets[i] * ets[j], axis=0)
            k += 1
    gt_ref[NPAIR:PADPAIR, :] = jnp.zeros((PADPAIR - NPAIR, BLK), jnp.float32)
    o_ref[...] = (
        jnp.dot(w1_ref[...], ecat_ref[...], preferred_element_type=jnp.float32)
        + jnp.dot(w2_ref[...], gt_ref[...], preferred_element_type=jnp.float32)
        + bo_ref[...]
    )


def _batch_compute(g3, denseT, Wd, bd2, W1m, W2m, bo2):
    return pl.pallas_call(
        _bat_body,
        grid=(B // BLK,),
        in_specs=[
            pl.BlockSpec((NP, BLK, PW), lambda i: (0, i, 0)),
            pl.BlockSpec((ND, BLK), lambda i: (0, i)),
            pl.BlockSpec((ED, ND), lambda i: (0, 0)),
            pl.BlockSpec((ED, 1), lambda i: (0, 0)),
            pl.BlockSpec((ED, NFE * ED), lambda i: (0, 0)),
            pl.BlockSpec((ED, PADPAIR), lambda i: (0, 0)),
            pl.BlockSpec((ED, 1), lambda i: (0, 0)),
        ],
        out_specs=pl.BlockSpec((ED, BLK), lambda i: (0, i)),
        out_shape=jax.ShapeDtypeStruct((ED, B), jnp.float32),
        scratch_shapes=[
            pltpu.VMEM((NFE * ED, BLK), jnp.float32),
            pltpu.VMEM((PADPAIR, BLK), jnp.float32),
        ],
    )(g3, denseT, Wd, bd2, W1m, W2m, bo2)


def kernel(dense, id_list, offsets, tables, Wd, bd, Wp, bp, Wo, bo):
    ids = (id_list.astype(jnp.int32)) % MH
    idx3 = (
        jnp.arange(NP, dtype=jnp.int32)[:, None] * MH + ids[None, :]
    ).reshape(NW, NCH, CH)
    tablesT4 = jnp.transpose(tables, (0, 2, 1)).reshape(NP, 2, H, MH)
    tproj = _table_transform(tablesT4, Wp.reshape(NP, 2, ED, H),
                             bp.reshape(NP, 2, ED, 1))
    g = _sc_gather(tproj.reshape(NP * MH, PW), idx3)
    g3 = g.reshape(NP, B, PW)
    W1m = Wo[:, : NFE * ED]
    W2m = jnp.pad(Wo[:, NFE * ED:], ((0, 0), (0, PADPAIR - NPAIR)))
    outT = _batch_compute(g3, dense.T, Wd, bd[:, None], W1m, W2m, bo[:, None])
    return outT.T


# speedup vs baseline: 12.2077x; 1.0176x over previous
import functools

import numpy as np
import jax
import jax.numpy as jnp
from jax import lax
from jax.experimental import pallas as pl
from jax.experimental.pallas import tpu as pltpu
from jax.experimental.pallas import tpu_sc as plsc

B = 4096
ND = 13
NF = 26
NFE = NF + 1
MH = 10000
H = 160
ED = 64

NC, NS = 2, 16
NW = NC * NS
NP = NF // 2
PW = 2 * ED
NROWS = NP * B
CH = 128
NCH = NP


@functools.partial(
    pl.kernel,
    mesh=plsc.VectorSubcoreMesh(core_axis_name="c", subcore_axis_name="s"),
    out_type=jax.ShapeDtypeStruct((NROWS, PW), jnp.float32),
    scratch_types=[
        pltpu.VMEM((CH,), jnp.int32),
        pltpu.VMEM((CH,), jnp.int32),
        pltpu.VMEM((CH,), jnp.int32),
        pltpu.VMEM((CH, PW), jnp.float32),
        pltpu.VMEM((CH, PW), jnp.float32),
        pltpu.SemaphoreType.DMA,
        pltpu.SemaphoreType.DMA,
    ],
)
def _sc_gather(tab_ref, ids_ref, out_ref, ids_v, idxA, idxB, buf0, buf1,
               sem0, sem1):
    wid = lax.axis_index("s") * NC + lax.axis_index("c")
    pltpu.sync_copy(ids_ref.at[pl.ds(wid * CH, CH)], ids_v)
    idxs = (idxA, idxB)
    bufs = (buf0, buf1)
    sems = (sem0, sem1)
    cps = []
    for c in range(NCH):
        idxs[c % 2][...] = ids_v[...] + c * MH
        cps.append(pltpu.async_copy(tab_ref.at[idxs[c % 2]],
                                    bufs[c % 2], sems[c % 2]))
        if c >= 1:
            cps[c - 1].wait()
            pltpu.sync_copy(
                bufs[(c - 1) % 2],
                out_ref.at[pl.ds(((c - 1) * NW + wid) * CH, CH)])
    cps[NCH - 1].wait()
    pltpu.sync_copy(bufs[(NCH - 1) % 2],
                    out_ref.at[pl.ds(((NCH - 1) * NW + wid) * CH, CH)])


TBLK = 2048


def _tab_body(t_ref, w_ref, b_ref, o_ref):
    ones_row = jnp.ones((1, H), jnp.float32)
    halves = []
    for k in range(2):
        r = t_ref[0, k]
        n2 = jnp.dot(ones_row, r * r, preferred_element_type=jnp.float32)
        s = jnp.where(n2 > 1.0, lax.rsqrt(n2), 1.0)
        halves.append(
            jnp.dot(w_ref[0, k], r, preferred_element_type=jnp.float32) * s
            + b_ref[0, k]
        )
    o_ref[0] = jnp.transpose(jnp.concatenate(halves, axis=0))


def _table_transform(tablesT4, Wp4, bp4):
    return pl.pallas_call(
        _tab_body,
        grid=(NP, pl.cdiv(MH, TBLK)),
        in_specs=[
            pl.BlockSpec((1, 2, H, TBLK), lambda p, i: (p, 0, 0, i)),
            pl.BlockSpec((1, 2, ED, H), lambda p, i: (p, 0, 0, 0)),
            pl.BlockSpec((1, 2, ED, 1), lambda p, i: (p, 0, 0, 0)),
        ],
        out_specs=pl.BlockSpec((1, TBLK, PW), lambda p, i: (p, i, 0)),
        out_shape=jax.ShapeDtypeStruct((NP, MH, PW), jnp.float32),
    )(tablesT4, Wp4, bp4)


BLK = 256
NPAIR = NFE * (NFE - 1) // 2


def _bat_body(g_ref, dt_ref, wd_ref, bd_ref, wo_ref, bo_ref, o_ref,
              ecat_ref, gt_ref):
    e0t = (
        jnp.dot(wd_ref[...], dt_ref[...], preferred_element_type=jnp.float32)
        + bd_ref[...]
    )
    ets = [e0t]
    for p in range(NP):
        tp = jnp.transpose(g_ref[p])
        ets.append(tp[:ED])
        ets.append(tp[ED:])
    for f in range(NFE):
        ecat_ref[f * ED:(f + 1) * ED, :] = ets[f]
    k = 0
    for i in range(NFE):
        for j in range(i + 1, NFE):
            gt_ref[k, :] = jnp.sum(ets[i] * ets[j], axis=0)
            k += 1
    w1 = wo_ref[:, : NFE * ED]
    w2 = wo_ref[:, NFE * ED:]
    o_ref[...] = (
        jnp.dot(w1, ecat_ref[...], preferred_element_type=jnp.float32)
        + jnp.dot(w2, gt_ref[...], preferred_element_type=jnp.float32)
        + bo_ref[...]
    )


def _batch_compute(g3, denseT, Wd, bd2, Wo, bo2):
    return pl.pallas_call(
        _bat_body,
        grid=(B // BLK,),
        in_specs=[
            pl.BlockSpec((NP, BLK, PW), lambda i: (0, i, 0)),
            pl.BlockSpec((ND, BLK), lambda i: (0, i)),
            pl.BlockSpec((ED, ND), lambda i: (0, 0)),
            pl.BlockSpec((ED, 1), lambda i: (0, 0)),
            pl.BlockSpec((ED, NFE * ED + NPAIR), lambda i: (0, 0)),
            pl.BlockSpec((ED, 1), lambda i: (0, 0)),
        ],
        out_specs=pl.BlockSpec((ED, BLK), lambda i: (0, i)),
        out_shape=jax.ShapeDtypeStruct((ED, B), jnp.float32),
        scratch_shapes=[
            pltpu.VMEM((NFE * ED, BLK), jnp.float32),
            pltpu.VMEM((NPAIR, BLK), jnp.float32),
        ],
    )(g3, denseT, Wd, bd2, Wo, bo2)


def kernel(dense, id_list, offsets, tables, Wd, bd, Wp, bp, Wo, bo):
    ids = (id_list.astype(jnp.int32)) % MH
    tablesT4 = jnp.transpose(tables, (0, 2, 1)).reshape(NP, 2, H, MH)
    tproj = _table_transform(tablesT4, Wp.reshape(NP, 2, ED, H),
                             bp.reshape(NP, 2, ED, 1))
    g = _sc_gather(tproj.reshape(NP * MH, PW), ids)
    g3 = g.reshape(NP, B, PW)
    outT = _batch_compute(g3, dense.T, Wd, bd[:, None], Wo, bo[:, None])
    return outT.T
